# baseline (speedup 1.0000x reference)
import functools

import jax
import jax.numpy as jnp
from jax import lax
from jax.experimental import pallas as pl
from jax.experimental.pallas import tpu as pltpu
from jax.experimental.pallas import tpu_sc as plsc

_N = 10000
_E = 320000
_D = 128
_EPS = 1e-5

_NC = 2
_NS = 16
_NW = _NC * _NS
_EPW = _E // _NW
_C = 128
_NFULL = _EPW // _C
_TAIL = _EPW - _NFULL * _C

_RPT = 624
_REM = _N - _NS * _RPT
_HALF = _N // 2
_HCH = 2000

_mesh = plsc.VectorSubcoreMesh(core_axis_name="c", subcore_axis_name="s")


def _copy_rows(src_get, dst_get, cnt):
    nfull = cnt // 128
    for k in range(nfull):
        pltpu.sync_copy(src_get(k * 128, 128), dst_get(k * 128, 128))
    rem = cnt - nfull * 128
    if rem:
        pltpu.sync_copy(src_get(nfull * 128, rem), dst_get(nfull * 128, rem))


@functools.partial(
    pl.kernel,
    out_type=jax.ShapeDtypeStruct((_NW * _N,), jnp.int32),
    mesh=_mesh,
    compiler_params=pltpu.CompilerParams(needs_layout_passes=False),
    scratch_types=[
        pltpu.VMEM((16, _HALF), jnp.int32),
        pltpu.VMEM((_HCH,), jnp.int32),
        pltpu.VMEM((_HALF,), jnp.int32),
    ],
)
def _deg_kernel(dst_hbm, out_hbm, histL, dbuf, red):
    c = lax.axis_index("c")
    s = lax.axis_index("s")
    w = s * _NC + c
    lanes = lax.broadcasted_iota(jnp.int32, (16,), 0)
    zeros16 = jnp.zeros((16,), jnp.int32)
    ones16 = jnp.ones((16,), jnp.int32)

    for p in range(_N // _HALF):
        lo = p * _HALF

        def _z(i, carry):
            r = i // (_HALF // 16)
            col = (i % (_HALF // 16)) * 16
            histL[r, pl.ds(col, 16)] = zeros16
            return carry

        lax.fori_loop(0, 16 * (_HALF // 16), _z, 0, unroll=8)

        def _round(rr, carry):
            off = pl.multiple_of(w * _EPW + rr * _HCH, 8)
            pltpu.sync_copy(dst_hbm.at[pl.ds(off, _HCH)], dbuf)

            def _step(kk, carry2):
                idx = dbuf[pl.ds(kk * 16, 16)]
                m = (idx >= lo) & (idx < lo + _HALF)
                il = jnp.clip(idx - lo, 0, _HALF - 1)
                v = plsc.load_gather(histL, [lanes, il], mask=m)
                plsc.store_scatter(histL, [lanes, il], v + ones16, mask=m)
                return carry2

            lax.fori_loop(0, _HCH // 16, _step, 0)
            return carry

        lax.fori_loop(0, _EPW // _HCH, _round, 0)

        def _red(j, carry):
            col = j * 16
            acc16 = zeros16
            for r in range(16):
                acc16 = acc16 + histL[r, pl.ds(col, 16)]
            red[pl.ds(col, 16)] = acc16
            return carry

        lax.fori_loop(0, _HALF // 16, _red, 0)
        obase = pl.multiple_of(w * _N + lo, 8)
        pltpu.sync_copy(red, out_hbm.at[pl.ds(obase, _HALF)])


_BCH = 16
_BE = _BCH * _C
_BATCHES = [_BCH] * (_NFULL // _BCH)
if _NFULL % _BCH:
    _BATCHES.append(_NFULL % _BCH)


@functools.partial(
    pl.kernel,
    out_type=jax.ShapeDtypeStruct((_NC * _N, _D), jnp.float32),
    mesh=_mesh,
    scratch_types=[
        pltpu.VMEM_SHARED((_N, _D), jnp.float32),
        pltpu.VMEM((_C, _D), jnp.float32),
        pltpu.VMEM((_C, _D), jnp.float32),
        pltpu.VMEM((_C,), jnp.int32),
        pltpu.VMEM((_C,), jnp.int32),
        pltpu.VMEM((_C,), jnp.int32),
        pltpu.VMEM((_C,), jnp.int32),
        pltpu.VMEM((_BE,), jnp.int32),
        pltpu.VMEM((_BE,), jnp.int32),
        pltpu.VMEM((_BE,), jnp.int32),
        pltpu.VMEM((_BE,), jnp.int32),
        pltpu.VMEM((_TAIL, _D), jnp.float32),
        pltpu.VMEM((_TAIL,), jnp.int32),
        pltpu.VMEM((_TAIL,), jnp.int32),
        pltpu.SemaphoreType.DMA,
        pltpu.SemaphoreType.DMA,
        pltpu.SemaphoreType.DMA,
        pltpu.SemaphoreType.DMA,
        pltpu.SemaphoreType.DMA,
        pltpu.SemaphoreType.DMA,
    ],
)
def _edge_kernel(hws_hbm, src_hbm, dst_hbm, out_hbm,
                 acc, rows0, rows1, isrc0, isrc1, idst0, idst1,
                 stSA, stDA, stSB, stDB, rows_t, isrc_t, idst_t,
                 sg0, sg1, ss0, ss1, stA, stB):
    c = lax.axis_index("c")
    s = lax.axis_index("s")
    w = s * _NC + c

    zv = jnp.zeros((16,), jnp.float32)

    def _zrow(i, carry):
        r = i // (_D // 16)
        col = (i % (_D // 16)) * 16
        rows0[r, pl.ds(col, 16)] = zv
        return carry

    lax.fori_loop(0, _C * _D // 16, _zrow, 0, unroll=8)

    base = pl.multiple_of(s * _RPT, 8)
    _copy_rows(lambda o, n: rows0.at[pl.ds(0, n)],
               lambda o, n: acc.at[pl.ds(base + o, n)], _RPT)

    @pl.when(s == _NS - 1)
    def _():
        pltpu.sync_copy(rows0.at[pl.ds(0, _REM)],
                        acc.at[pl.ds(_NS * _RPT, _REM)])

    plsc.subcore_barrier()

    ebase = w * _EPW
    stg = [(stSA, stDA, stA), (stSB, stDB, stB)]

    def _stage(b, nch, sync):
        off = pl.multiple_of(ebase + b * _BE, 8)
        sS, sD, sem = stg[b % 2]
        if sync:
            pltpu.sync_copy(src_hbm.at[pl.ds(off, nch * _C)],
                            sS.at[pl.ds(0, nch * _C)])
            pltpu.sync_copy(dst_hbm.at[pl.ds(off, nch * _C)],
                            sD.at[pl.ds(0, nch * _C)])
        else:
            pltpu.async_copy(src_hbm.at[pl.ds(off, nch * _C)],
                             sS.at[pl.ds(0, nch * _C)], sem)
            pltpu.async_copy(dst_hbm.at[pl.ds(off, nch * _C)],
                             sD.at[pl.ds(0, nch * _C)], sem)

    def _stage_wait(b, nch):
        sS, sD, sem = stg[b % 2]
        pltpu.make_async_copy(src_hbm.at[pl.ds(0, nch * _C)],
                              sS.at[pl.ds(0, nch * _C)], sem).wait()
        pltpu.make_async_copy(dst_hbm.at[pl.ds(0, nch * _C)],
                              sD.at[pl.ds(0, nch * _C)], sem).wait()

    def _copy_idx(sS, sD, off, isrc, idst):
        for q in range(_C // 16):
            isrc[pl.ds(q * 16, 16)] = sS[pl.ds(off + q * 16, 16)]
            idst[pl.ds(q * 16, 16)] = sD[pl.ds(off + q * 16, 16)]

    def _pair(sS, sD, off, wait_scatter):
        if wait_scatter:
            pltpu.make_async_copy(rows0, acc.at[idst0], ss0).wait()
            pltpu.make_async_copy(rows1, acc.at[idst1], ss1).wait()
        _copy_idx(sS, sD, off, isrc0, idst0)
        _copy_idx(sS, sD, off + _C, isrc1, idst1)
        pltpu.async_copy(hws_hbm.at[isrc0], rows0, sg0)
        pltpu.async_copy(hws_hbm.at[isrc1], rows1, sg1)
        pltpu.make_async_copy(hws_hbm.at[isrc0], rows0, sg0).wait()
        pltpu.async_copy(rows0, acc.at[idst0], ss0, add=True)
        pltpu.make_async_copy(hws_hbm.at[isrc1], rows1, sg1).wait()
        pltpu.async_copy(rows1, acc.at[idst1], ss1, add=True)

    _stage(0, _BATCHES[0], sync=True)
    if len(_BATCHES) > 1:
        _stage(1, _BATCHES[1], sync=False)

    for b, nch in enumerate(_BATCHES):
        sS, sD, _ = stg[b % 2]
        if b > 0:
            _stage_wait(b, nch)
        if b + 1 < len(_BATCHES) and b > 0:
            _stage(b + 1, _BATCHES[b + 1], sync=False)
        if b == 0:
            _pair(sS, sD, 0, wait_scatter=False)
            start = 1
        else:
            start = 0

        def _body(step, carry):
            _pair(sS, sD, step * 2 * _C, wait_scatter=True)
            return carry

        lax.fori_loop(start, nch // 2, _body, 0)

    pltpu.make_async_copy(rows0, acc.at[idst0], ss0).wait()
    pltpu.make_async_copy(rows1, acc.at[idst1], ss1).wait()

    if _TAIL:
        off = pl.multiple_of(ebase + _NFULL * _C, 8)
        pltpu.sync_copy(src_hbm.at[pl.ds(off, _TAIL)], isrc_t)
        pltpu.sync_copy(dst_hbm.at[pl.ds(off, _TAIL)], idst_t)
        pltpu.async_copy(hws_hbm.at[isrc_t], rows_t, sg0).wait()
        pltpu.sync_copy(rows_t, acc.at[idst_t], add=True)

    plsc.subcore_barrier()

    obase = pl.multiple_of(c * _N + s * _RPT, 8)
    _copy_rows(lambda o, n: acc.at[pl.ds(base + o, n)],
               lambda o, n: out_hbm.at[pl.ds(obase + o, n)], _RPT)

    @pl.when(s == _NS - 1)
    def _():
        o2 = pl.multiple_of(c * _N + _NS * _RPT, 8)
        pltpu.sync_copy(acc.at[pl.ds(_NS * _RPT, _REM)],
                        out_hbm.at[pl.ds(o2, _REM)])


_R = 1000


def _prep_body(cnt_ref, x_ref, w1_ref, hws_ref, dinv_ref):
    deg = jnp.sum(cnt_ref[...].astype(jnp.float32), axis=1, keepdims=True) + 1.0
    dinv = lax.rsqrt(deg)
    hws_ref[...] = jnp.dot(x_ref[...], w1_ref[...],
                           preferred_element_type=jnp.float32) * dinv
    dinv_ref[...] = dinv


def _prep(cnt, x, W1):
    return pl.pallas_call(
        _prep_body,
        grid=(_N // _R,),
        in_specs=[
            pl.BlockSpec((_R, _NW), lambda i: (i, 0)),
            pl.BlockSpec((_R, _D), lambda i: (i, 0)),
            pl.BlockSpec((_D, _D), lambda i: (0, 0)),
        ],
        out_specs=[
            pl.BlockSpec((_R, _D), lambda i: (i, 0)),
            pl.BlockSpec((_R, 1), lambda i: (i, 0)),
        ],
        out_shape=[
            jax.ShapeDtypeStruct((_N, _D), jnp.float32),
            jax.ShapeDtypeStruct((_N, 1), jnp.float32),
        ],
    )(cnt, x, W1)


def _norm_elu(p_ref, hws_ref, dinv_ref, b_ref, g_ref, be_ref, res_ref):
    dinv = dinv_ref[...]
    agg = (p_ref[0] + p_ref[1] + hws_ref[...]) * dinv + b_ref[...]
    mu = jnp.mean(agg, axis=1, keepdims=True)
    d = agg - mu
    var = jnp.mean(d * d, axis=1, keepdims=True)
    hn = d * lax.rsqrt(var + _EPS) * g_ref[...] + be_ref[...]
    hn = jnp.where(hn > 0, hn, jnp.exp(hn) - 1.0)
    return hn + res_ref[...]


def _mid_body(p_ref, hws_ref, dinv_ref, res_ref, b_ref, g_ref, be_ref, w2_ref,
              h1_ref, hws2_ref):
    h1 = _norm_elu(p_ref, hws_ref, dinv_ref, b_ref, g_ref, be_ref, res_ref)
    h1_ref[...] = h1
    hws2_ref[...] = jnp.dot(h1, w2_ref[...],
                            preferred_element_type=jnp.float32) * dinv_ref[...]


def _mid(p, hws1, dinv, x, b1, g1, be1, W2):
    return pl.pallas_call(
        _mid_body,
        grid=(_N // _R,),
        in_specs=[
            pl.BlockSpec((_NC, _R, _D), lambda i: (0, i, 0)),
            pl.BlockSpec((_R, _D), lambda i: (i, 0)),
            pl.BlockSpec((_R, 1), lambda i: (i, 0)),
            pl.BlockSpec((_R, _D), lambda i: (i, 0)),
            pl.BlockSpec((1, _D), lambda i: (0, 0)),
            pl.BlockSpec((1, _D), lambda i: (0, 0)),
            pl.BlockSpec((1, _D), lambda i: (0, 0)),
            pl.BlockSpec((_D, _D), lambda i: (0, 0)),
        ],
        out_specs=[
            pl.BlockSpec((_R, _D), lambda i: (i, 0)),
            pl.BlockSpec((_R, _D), lambda i: (i, 0)),
        ],
        out_shape=[
            jax.ShapeDtypeStruct((_N, _D), jnp.float32),
            jax.ShapeDtypeStruct((_N, _D), jnp.float32),
        ],
    )(p, hws1, dinv, x, b1, g1, be1, W2)


def _fin_body(p_ref, hws_ref, dinv_ref, res_ref, b_ref, g_ref, be_ref, out_ref):
    out_ref[...] = _norm_elu(p_ref, hws_ref, dinv_ref, b_ref, g_ref, be_ref,
                             res_ref)


def _fin(p, hws2, dinv, h1, b2, g2, be2):
    return pl.pallas_call(
        _fin_body,
        grid=(_N // _R,),
        in_specs=[
            pl.BlockSpec((_NC, _R, _D), lambda i: (0, i, 0)),
            pl.BlockSpec((_R, _D), lambda i: (i, 0)),
            pl.BlockSpec((_R, 1), lambda i: (i, 0)),
            pl.BlockSpec((_R, _D), lambda i: (i, 0)),
            pl.BlockSpec((1, _D), lambda i: (0, 0)),
            pl.BlockSpec((1, _D), lambda i: (0, 0)),
            pl.BlockSpec((1, _D), lambda i: (0, 0)),
        ],
        out_specs=pl.BlockSpec((_R, _D), lambda i: (i, 0)),
        out_shape=jax.ShapeDtypeStruct((_N, _D), jnp.float32),
    )(p, hws2, dinv, h1, b2, g2, be2)


def kernel(x, edge_index, W1, b1, g1, be1, W2, b2, g2, be2):
    src = edge_index[0]
    dst = edge_index[1]

    cnts = _deg_kernel(dst)
    cntT = cnts.reshape(_NW, _N).T

    hws1, dinv = _prep(cntT, x, W1)
    p1 = _edge_kernel(hws1, src, dst).reshape(_NC, _N, _D)
    h1, hws2 = _mid(p1, hws1, dinv, x, b1.reshape(1, _D), g1.reshape(1, _D),
                    be1.reshape(1, _D), W2)
    p2 = _edge_kernel(hws2, src, dst).reshape(_NC, _N, _D)
    out = _fin(p2, hws2, dinv, h1, b2.reshape(1, _D), g2.reshape(1, _D),
               be2.reshape(1, _D))
    return out


# speedup vs baseline: 1.0222x; 1.0222x over previous
import functools

import jax
import jax.numpy as jnp
from jax import lax
from jax.experimental import pallas as pl
from jax.experimental.pallas import tpu as pltpu
from jax.experimental.pallas import tpu_sc as plsc

_N = 10000
_E = 320000
_D = 128
_EPS = 1e-5

_NC = 2
_NS = 16
_NW = _NC * _NS
_EPW = _E // _NW
_C = 128
_NFULL = _EPW // _C
_TAIL = _EPW - _NFULL * _C

_RPT = 624
_REM = _N - _NS * _RPT
_HALF = _N // 2
_HCH = 2000

_mesh = plsc.VectorSubcoreMesh(core_axis_name="c", subcore_axis_name="s")


def _copy_rows(src_get, dst_get, cnt):
    nfull = cnt // 128
    for k in range(nfull):
        pltpu.sync_copy(src_get(k * 128, 128), dst_get(k * 128, 128))
    rem = cnt - nfull * 128
    if rem:
        pltpu.sync_copy(src_get(nfull * 128, rem), dst_get(nfull * 128, rem))


@functools.partial(
    pl.kernel,
    out_type=jax.ShapeDtypeStruct((_NW * _N,), jnp.int32),
    mesh=_mesh,
    compiler_params=pltpu.CompilerParams(needs_layout_passes=False),
    scratch_types=[
        pltpu.VMEM((16, _HALF), jnp.int32),
        pltpu.VMEM((_HCH,), jnp.int32),
        pltpu.VMEM((_HALF,), jnp.int32),
    ],
)
def _deg_kernel(dst_hbm, out_hbm, histL, dbuf, red):
    c = lax.axis_index("c")
    s = lax.axis_index("s")
    w = s * _NC + c
    lanes = lax.broadcasted_iota(jnp.int32, (16,), 0)
    zeros16 = jnp.zeros((16,), jnp.int32)
    ones16 = jnp.ones((16,), jnp.int32)

    for p in range(_N // _HALF):
        lo = p * _HALF

        def _z(i, carry):
            r = i // (_HALF // 16)
            col = (i % (_HALF // 16)) * 16
            histL[r, pl.ds(col, 16)] = zeros16
            return carry

        lax.fori_loop(0, 16 * (_HALF // 16), _z, 0, unroll=8)

        def _round(rr, carry):
            off = pl.multiple_of(w * _EPW + rr * _HCH, 8)
            pltpu.sync_copy(dst_hbm.at[pl.ds(off, _HCH)], dbuf)

            def _step(kk, carry2):
                idx = dbuf[pl.ds(kk * 16, 16)]
                m = (idx >= lo) & (idx < lo + _HALF)
                il = jnp.clip(idx - lo, 0, _HALF - 1)
                v = plsc.load_gather(histL, [lanes, il], mask=m)
                plsc.store_scatter(histL, [lanes, il], v + ones16, mask=m)
                return carry2

            lax.fori_loop(0, _HCH // 16, _step, 0)
            return carry

        lax.fori_loop(0, _EPW // _HCH, _round, 0)

        def _red(j, carry):
            col = j * 16
            acc16 = zeros16
            for r in range(16):
                acc16 = acc16 + histL[r, pl.ds(col, 16)]
            red[pl.ds(col, 16)] = acc16
            return carry

        lax.fori_loop(0, _HALF // 16, _red, 0)
        obase = pl.multiple_of(w * _N + lo, 8)
        pltpu.sync_copy(red, out_hbm.at[pl.ds(obase, _HALF)])


@functools.partial(
    pl.kernel,
    out_type=jax.ShapeDtypeStruct((_NC * _N, _D), jnp.float32),
    mesh=_mesh,
    scratch_types=[
        pltpu.VMEM_SHARED((_N, _D), jnp.float32),
        pltpu.VMEM((_C, _D), jnp.float32),
        pltpu.VMEM((_C, _D), jnp.float32),
        pltpu.VMEM((_C,), jnp.int32),
        pltpu.VMEM((_C,), jnp.int32),
        pltpu.VMEM((_C,), jnp.int32),
        pltpu.VMEM((_C,), jnp.int32),
        pltpu.VMEM((_TAIL, _D), jnp.float32),
        pltpu.VMEM((_TAIL,), jnp.int32),
        pltpu.VMEM((_TAIL,), jnp.int32),
        pltpu.SemaphoreType.DMA,
        pltpu.SemaphoreType.DMA,
    ],
)
def _edge_kernel(hws_hbm, src_hbm, dst_hbm, out_hbm,
                 acc, rows0, rows1, isrc0, isrc1, idst0, idst1,
                 rows_t, isrc_t, idst_t, sem0, sem1):
    c = lax.axis_index("c")
    s = lax.axis_index("s")
    w = s * _NC + c

    zv = jnp.zeros((16,), jnp.float32)

    def _zrow(i, carry):
        r = i // (_D // 16)
        col = (i % (_D // 16)) * 16
        rows0[r, pl.ds(col, 16)] = zv
        return carry

    lax.fori_loop(0, _C * _D // 16, _zrow, 0, unroll=8)

    base = pl.multiple_of(s * _RPT, 8)
    _copy_rows(lambda o, n: rows0.at[pl.ds(0, n)],
               lambda o, n: acc.at[pl.ds(base + o, n)], _RPT)

    @pl.when(s == _NS - 1)
    def _():
        pltpu.sync_copy(rows0.at[pl.ds(0, _REM)],
                        acc.at[pl.ds(_NS * _RPT, _REM)])

    plsc.subcore_barrier()

    ebase = w * _EPW

    def _load(j, isrc, idst):
        off = pl.multiple_of(ebase + j * _C, 8)
        pltpu.sync_copy(src_hbm.at[pl.ds(off, _C)], isrc)
        pltpu.sync_copy(dst_hbm.at[pl.ds(off, _C)], idst)

    _load(0, isrc0, idst0)
    pltpu.async_copy(hws_hbm.at[isrc0], rows0, sem0)

    def _pair(step, carry):
        j0 = step * 2
        _load(j0 + 1, isrc1, idst1)
        pltpu.async_copy(hws_hbm.at[isrc1], rows1, sem1)
        pltpu.make_async_copy(hws_hbm.at[isrc0], rows0, sem0).wait()
        pltpu.sync_copy(rows0, acc.at[idst0], add=True)
        _load(j0 + 2, isrc0, idst0)
        pltpu.async_copy(hws_hbm.at[isrc0], rows0, sem0)
        pltpu.make_async_copy(hws_hbm.at[isrc1], rows1, sem1).wait()
        pltpu.sync_copy(rows1, acc.at[idst1], add=True)
        return carry

    lax.fori_loop(0, (_NFULL - 2) // 2, _pair, 0)

    _load(_NFULL - 1, isrc1, idst1)
    pltpu.async_copy(hws_hbm.at[isrc1], rows1, sem1)
    pltpu.make_async_copy(hws_hbm.at[isrc0], rows0, sem0).wait()
    pltpu.sync_copy(rows0, acc.at[idst0], add=True)
    pltpu.make_async_copy(hws_hbm.at[isrc1], rows1, sem1).wait()
    pltpu.sync_copy(rows1, acc.at[idst1], add=True)

    if _TAIL:
        off = pl.multiple_of(ebase + _NFULL * _C, 8)
        pltpu.sync_copy(src_hbm.at[pl.ds(off, _TAIL)], isrc_t)
        pltpu.sync_copy(dst_hbm.at[pl.ds(off, _TAIL)], idst_t)
        pltpu.async_copy(hws_hbm.at[isrc_t], rows_t, sem0).wait()
        pltpu.sync_copy(rows_t, acc.at[idst_t], add=True)

    plsc.subcore_barrier()

    obase = pl.multiple_of(c * _N + s * _RPT, 8)
    _copy_rows(lambda o, n: acc.at[pl.ds(base + o, n)],
               lambda o, n: out_hbm.at[pl.ds(obase + o, n)], _RPT)

    @pl.when(s == _NS - 1)
    def _():
        o2 = pl.multiple_of(c * _N + _NS * _RPT, 8)
        pltpu.sync_copy(acc.at[pl.ds(_NS * _RPT, _REM)],
                        out_hbm.at[pl.ds(o2, _REM)])


_R = 1000


def _prep_body(cnt_ref, x_ref, w1_ref, hws_ref, dinv_ref):
    deg = jnp.sum(cnt_ref[...].astype(jnp.float32), axis=1, keepdims=True) + 1.0
    dinv = lax.rsqrt(deg)
    hws_ref[...] = jnp.dot(x_ref[...], w1_ref[...],
                           preferred_element_type=jnp.float32) * dinv
    dinv_ref[...] = dinv


def _prep(cnt, x, W1):
    return pl.pallas_call(
        _prep_body,
        grid=(_N // _R,),
        in_specs=[
            pl.BlockSpec((_R, _NW), lambda i: (i, 0)),
            pl.BlockSpec((_R, _D), lambda i: (i, 0)),
            pl.BlockSpec((_D, _D), lambda i: (0, 0)),
        ],
        out_specs=[
            pl.BlockSpec((_R, _D), lambda i: (i, 0)),
            pl.BlockSpec((_R, 1), lambda i: (i, 0)),
        ],
        out_shape=[
            jax.ShapeDtypeStruct((_N, _D), jnp.float32),
            jax.ShapeDtypeStruct((_N, 1), jnp.float32),
        ],
    )(cnt, x, W1)


def _norm_elu(p_ref, hws_ref, dinv_ref, b_ref, g_ref, be_ref, res_ref):
    dinv = dinv_ref[...]
    agg = (p_ref[0] + p_ref[1] + hws_ref[...]) * dinv + b_ref[...]
    mu = jnp.mean(agg, axis=1, keepdims=True)
    d = agg - mu
    var = jnp.mean(d * d, axis=1, keepdims=True)
    hn = d * lax.rsqrt(var + _EPS) * g_ref[...] + be_ref[...]
    hn = jnp.where(hn > 0, hn, jnp.exp(hn) - 1.0)
    return hn + res_ref[...]


def _mid_body(p_ref, hws_ref, dinv_ref, res_ref, b_ref, g_ref, be_ref, w2_ref,
              h1_ref, hws2_ref):
    h1 = _norm_elu(p_ref, hws_ref, dinv_ref, b_ref, g_ref, be_ref, res_ref)
    h1_ref[...] = h1
    hws2_ref[...] = jnp.dot(h1, w2_ref[...],
                            preferred_element_type=jnp.float32) * dinv_ref[...]


def _mid(p, hws1, dinv, x, b1, g1, be1, W2):
    return pl.pallas_call(
        _mid_body,
        grid=(_N // _R,),
        in_specs=[
            pl.BlockSpec((_NC, _R, _D), lambda i: (0, i, 0)),
            pl.BlockSpec((_R, _D), lambda i: (i, 0)),
            pl.BlockSpec((_R, 1), lambda i: (i, 0)),
            pl.BlockSpec((_R, _D), lambda i: (i, 0)),
            pl.BlockSpec((1, _D), lambda i: (0, 0)),
            pl.BlockSpec((1, _D), lambda i: (0, 0)),
            pl.BlockSpec((1, _D), lambda i: (0, 0)),
            pl.BlockSpec((_D, _D), lambda i: (0, 0)),
        ],
        out_specs=[
            pl.BlockSpec((_R, _D), lambda i: (i, 0)),
            pl.BlockSpec((_R, _D), lambda i: (i, 0)),
        ],
        out_shape=[
            jax.ShapeDtypeStruct((_N, _D), jnp.float32),
            jax.ShapeDtypeStruct((_N, _D), jnp.float32),
        ],
    )(p, hws1, dinv, x, b1, g1, be1, W2)


def _fin_body(p_ref, hws_ref, dinv_ref, res_ref, b_ref, g_ref, be_ref, out_ref):
    out_ref[...] = _norm_elu(p_ref, hws_ref, dinv_ref, b_ref, g_ref, be_ref,
                             res_ref)


def _fin(p, hws2, dinv, h1, b2, g2, be2):
    return pl.pallas_call(
        _fin_body,
        grid=(_N // _R,),
        in_specs=[
            pl.BlockSpec((_NC, _R, _D), lambda i: (0, i, 0)),
            pl.BlockSpec((_R, _D), lambda i: (i, 0)),
            pl.BlockSpec((_R, 1), lambda i: (i, 0)),
            pl.BlockSpec((_R, _D), lambda i: (i, 0)),
            pl.BlockSpec((1, _D), lambda i: (0, 0)),
            pl.BlockSpec((1, _D), lambda i: (0, 0)),
            pl.BlockSpec((1, _D), lambda i: (0, 0)),
        ],
        out_specs=pl.BlockSpec((_R, _D), lambda i: (i, 0)),
        out_shape=jax.ShapeDtypeStruct((_N, _D), jnp.float32),
    )(p, hws2, dinv, h1, b2, g2, be2)


def kernel(x, edge_index, W1, b1, g1, be1, W2, b2, g2, be2):
    src = edge_index[0]
    dst = edge_index[1]

    cnts = _deg_kernel(dst)
    cntT = cnts.reshape(_NW, _N).T

    hws1, dinv = _prep(cntT, x, W1)
    p1 = _edge_kernel(hws1, src, dst).reshape(_NC, _N, _D)
    h1, hws2 = _mid(p1, hws1, dinv, x, b1.reshape(1, _D), g1.reshape(1, _D),
                    be1.reshape(1, _D), W2)
    p2 = _edge_kernel(hws2, src, dst).reshape(_NC, _N, _D)
    out = _fin(p2, hws2, dinv, h1, b2.reshape(1, _D), g2.reshape(1, _D),
               be2.reshape(1, _D))
    return out
